# baseline (speedup 1.0000x reference)
import jax
import jax.numpy as jnp
from jax.experimental import pallas as pl
from jax.experimental.pallas import tpu as pltpu

K = 16
NP = 16384
NL = 2048
FDIM = 512
ROWS = 128
NBLK = NL // ROWS
NCH = 128
CW = 128
NSTORE = 5
PBLK = 1024


def _select_body(ligt_ref, px8_ref, ligf_ref, counts_ref, ligctx_ref,
                 sel_ref):
    yt = ligt_ref[...]
    x = px8_ref[...]
    y2 = jnp.sum(yt * yt, axis=0, keepdims=True)
    x2 = jnp.sum(x * x, axis=1, keepdims=True)
    d2 = y2 + x2 - 2.0 * jnp.dot(x, yt, preferred_element_type=jnp.float32)
    dt = jnp.maximum(d2, 0.0)

    big = jnp.int32(1 << 30)
    inf = jnp.float32(jnp.inf)

    dtc = dt.reshape(NCH, CW, ROWS)
    piota = jax.lax.broadcasted_iota(jnp.int32, (NCH, CW, ROWS), 1)
    c = dtc
    vs, ps = [], []
    for _ in range(NSTORE):
        mn = jnp.min(c, axis=1, keepdims=True)
        hit = c == mn
        pos = jnp.min(jnp.where(hit, piota, big), axis=1, keepdims=True)
        vs.append(mn[:, 0, :])
        ps.append(pos[:, 0, :])
        c = jnp.where(hit & (piota == pos), inf, c)
    chiota = jax.lax.broadcasted_iota(jnp.int32, (NCH, ROWS), 0)
    v = list(vs)
    g = [p + chiota * CW for p in ps]

    sel_ids = []
    flags = jnp.zeros((NCH, ROWS), jnp.bool_)
    for t in range(K):
        mn = jnp.min(v[0], axis=0, keepdims=True)
        eq = v[0] == mn
        gsel = jnp.min(jnp.where(eq, g[0], big), axis=0, keepdims=True)
        sel_ids.append(gsel)
        win = eq & (g[0] == gsel)
        if t < K - 1:
            flags = flags | (win & jnp.isinf(v[1]))
        for s in range(NSTORE - 1):
            v[s] = jnp.where(win, v[s + 1], v[s])
            g[s] = jnp.where(win, g[s + 1], g[s])
        v[NSTORE - 1] = jnp.where(win, inf, v[NSTORE - 1])
    sel_ref[...] = jnp.concatenate(sel_ids, axis=0)

    overflow = jnp.max(flags.astype(jnp.int32)) > 0

    @pl.when(overflow)
    def _():
        riota = jax.lax.broadcasted_iota(jnp.int32, (NP, ROWS), 0)

        def step(t, cc):
            mn2 = jnp.min(cc, axis=0, keepdims=True)
            eq2 = cc == mn2
            idx2 = jnp.min(jnp.where(eq2, riota, big), axis=0,
                           keepdims=True)
            sel_ref[pl.ds(t, 1), :] = idx2
            return jnp.where(eq2 & (riota == idx2), inf, cc)

        jax.lax.fori_loop(0, K, step, dt)

    sel = sel_ref[...]

    sc = sel // CW
    sl = sel - sc * CW
    aoh = (sc[:, :, None] == jax.lax.broadcasted_iota(
        jnp.int32, (K, ROWS, NCH), 2)).astype(jnp.float32)
    boh = (sl[:, :, None] == jax.lax.broadcasted_iota(
        jnp.int32, (K, ROWS, CW), 2)).astype(jnp.float32)
    cpart = jax.lax.dot_general(
        aoh.reshape(K * ROWS, NCH), boh.reshape(K * ROWS, CW),
        dimension_numbers=(((0,), (0,)), ((), ())),
        preferred_element_type=jnp.float32)
    counts_ref[...] = cpart[None]
    ligctx_ref[...] = jnp.sum(ligf_ref[...], axis=0).reshape(1, 1, FDIM)


def _reduce_body(counts_ref, pf_ref, out_ref):
    i = pl.program_id(0)
    w = jnp.sum(counts_ref[...], axis=0)
    f = pf_ref[...]
    part = jnp.sum(w[:, None] * f, axis=0)

    @pl.when(i == 0)
    def _():
        out_ref[...] = part

    @pl.when(i != 0)
    def _():
        out_ref[...] += part


@jax.jit
def kernel(protein_pos, protein_atom_feature, ligand_pos, ligand_atom_feature):
    pos8 = jnp.pad(protein_pos, ((0, 0), (0, 5)))
    ligt = jnp.pad(ligand_pos, ((0, 0), (0, 5))).T

    counts3d, ligctx2d = pl.pallas_call(
        _select_body,
        grid=(NBLK,),
        in_specs=[
            pl.BlockSpec((8, ROWS), lambda i: (0, i)),
            pl.BlockSpec((NP, 8), lambda i: (0, 0)),
            pl.BlockSpec((ROWS, FDIM), lambda i: (i, 0)),
        ],
        out_specs=[
            pl.BlockSpec((1, NCH, CW), lambda i: (i, 0, 0)),
            pl.BlockSpec((1, 1, FDIM), lambda i: (i, 0, 0)),
        ],
        out_shape=[
            jax.ShapeDtypeStruct((NBLK, NCH, CW), jnp.float32),
            jax.ShapeDtypeStruct((NBLK, 1, FDIM), jnp.float32),
        ],
        scratch_shapes=[pltpu.VMEM((K, ROWS), jnp.int32)],
        compiler_params=pltpu.CompilerParams(
            dimension_semantics=("parallel",)),
    )(ligt, pos8, ligand_atom_feature)

    counts = counts3d.reshape(NBLK, NP)
    ligctx = jnp.sum(ligctx2d[:, 0, :], axis=0)

    psum = pl.pallas_call(
        _reduce_body,
        grid=(NP // PBLK,),
        in_specs=[
            pl.BlockSpec((NBLK, PBLK), lambda i: (0, i)),
            pl.BlockSpec((PBLK, FDIM), lambda i: (i, 0)),
        ],
        out_specs=pl.BlockSpec((FDIM,), lambda i: (0,)),
        out_shape=jax.ShapeDtypeStruct((FDIM,), jnp.float32),
    )(counts, protein_atom_feature)

    return jnp.concatenate([ligctx, psum * (1.0 / NL)])


# speedup vs baseline: 1.1127x; 1.1127x over previous
import jax
import jax.numpy as jnp
from jax.experimental import pallas as pl
from jax.experimental.pallas import tpu as pltpu

K = 16
NP = 16384
NL = 2048
FDIM = 512
ROWS = 128
NBLK = NL // ROWS
NCH = 128
CW = 128
NSTORE = 5
PBLK = 1024


def _select_body(ligt_ref, px8_ref, ligf_ref, counts_ref, ligctx_ref,
                 sel_ref):
    yt = ligt_ref[...]
    x = px8_ref[...]
    y2 = jnp.sum(yt * yt, axis=0, keepdims=True)
    x2 = jnp.sum(x * x, axis=1, keepdims=True)
    d2 = y2 + x2 - 2.0 * jnp.dot(x, yt, preferred_element_type=jnp.float32)
    dt = jnp.maximum(d2, 0.0)

    big = jnp.int32(1 << 30)
    inf = jnp.float32(jnp.inf)

    dtc = dt.reshape(NCH, CW, ROWS)
    piota = jax.lax.broadcasted_iota(jnp.int32, (NCH, CW, ROWS), 1)
    c = dtc
    vs, ps = [], []
    for t in range(NSTORE):
        mn = jnp.min(c, axis=1, keepdims=True)
        hit = c == mn
        pos = jnp.min(jnp.where(hit, piota, big), axis=1, keepdims=True)
        vs.append(mn[:, 0, :])
        ps.append(pos[:, 0, :])
        if t < NSTORE - 1:
            c = jnp.where(piota == pos, inf, c)
    chiota = jax.lax.broadcasted_iota(jnp.int32, (NCH, ROWS), 0)
    v = list(vs)
    g = [p + chiota * CW for p in ps]

    sel_ids = []
    flags = jnp.zeros((NCH, ROWS), jnp.bool_)
    for t in range(K):
        mn = jnp.min(v[0], axis=0, keepdims=True)
        eq = v[0] == mn
        gsel = jnp.min(jnp.where(eq, g[0], big), axis=0, keepdims=True)
        sel_ids.append(gsel)
        win = g[0] == gsel
        if t < K - 1:
            flags = flags | (win & jnp.isinf(v[1]))
            for s in range(NSTORE - 1):
                v[s] = jnp.where(win, v[s + 1], v[s])
                g[s] = jnp.where(win, g[s + 1], g[s])
            v[NSTORE - 1] = jnp.where(win, inf, v[NSTORE - 1])
    sel_ref[...] = jnp.concatenate(sel_ids, axis=0)

    overflow = jnp.max(flags.astype(jnp.int32)) > 0

    @pl.when(overflow)
    def _():
        riota = jax.lax.broadcasted_iota(jnp.int32, (NP, ROWS), 0)

        def step(t, cc):
            mn2 = jnp.min(cc, axis=0, keepdims=True)
            eq2 = cc == mn2
            idx2 = jnp.min(jnp.where(eq2, riota, big), axis=0,
                           keepdims=True)
            sel_ref[pl.ds(t, 1), :] = idx2
            return jnp.where(eq2 & (riota == idx2), inf, cc)

        jax.lax.fori_loop(0, K, step, dt)

    sel = sel_ref[...]

    sc = sel // CW
    sl = sel - sc * CW
    aoh = (sc[:, :, None] == jax.lax.broadcasted_iota(
        jnp.int32, (K, ROWS, NCH), 2)).astype(jnp.float32)
    boh = (sl[:, :, None] == jax.lax.broadcasted_iota(
        jnp.int32, (K, ROWS, CW), 2)).astype(jnp.float32)
    cpart = jax.lax.dot_general(
        aoh.reshape(K * ROWS, NCH), boh.reshape(K * ROWS, CW),
        dimension_numbers=(((0,), (0,)), ((), ())),
        preferred_element_type=jnp.float32)
    counts_ref[...] = cpart[None]
    ligctx_ref[...] = jnp.sum(ligf_ref[...], axis=0).reshape(1, 1, FDIM)


def _reduce_body(counts_ref, pf_ref, out_ref):
    i = pl.program_id(0)
    w = jnp.sum(counts_ref[...], axis=0)
    f = pf_ref[...]
    part = jnp.sum(w[:, None] * f, axis=0)

    @pl.when(i == 0)
    def _():
        out_ref[...] = part

    @pl.when(i != 0)
    def _():
        out_ref[...] += part


@jax.jit
def kernel(protein_pos, protein_atom_feature, ligand_pos, ligand_atom_feature):
    pos8 = jnp.pad(protein_pos, ((0, 0), (0, 5)))
    ligt = jnp.pad(ligand_pos, ((0, 0), (0, 5))).T

    counts3d, ligctx2d = pl.pallas_call(
        _select_body,
        grid=(NBLK,),
        in_specs=[
            pl.BlockSpec((8, ROWS), lambda i: (0, i)),
            pl.BlockSpec((NP, 8), lambda i: (0, 0)),
            pl.BlockSpec((ROWS, FDIM), lambda i: (i, 0)),
        ],
        out_specs=[
            pl.BlockSpec((1, NCH, CW), lambda i: (i, 0, 0)),
            pl.BlockSpec((1, 1, FDIM), lambda i: (i, 0, 0)),
        ],
        out_shape=[
            jax.ShapeDtypeStruct((NBLK, NCH, CW), jnp.float32),
            jax.ShapeDtypeStruct((NBLK, 1, FDIM), jnp.float32),
        ],
        scratch_shapes=[pltpu.VMEM((K, ROWS), jnp.int32)],
        compiler_params=pltpu.CompilerParams(
            dimension_semantics=("parallel",)),
    )(ligt, pos8, ligand_atom_feature)

    counts = counts3d.reshape(NBLK, NP)
    ligctx = jnp.sum(ligctx2d[:, 0, :], axis=0)

    psum = pl.pallas_call(
        _reduce_body,
        grid=(NP // PBLK,),
        in_specs=[
            pl.BlockSpec((NBLK, PBLK), lambda i: (0, i)),
            pl.BlockSpec((PBLK, FDIM), lambda i: (i, 0)),
        ],
        out_specs=pl.BlockSpec((FDIM,), lambda i: (0,)),
        out_shape=jax.ShapeDtypeStruct((FDIM,), jnp.float32),
    )(counts, protein_atom_feature)

    return jnp.concatenate([ligctx, psum * (1.0 / NL)])
